# Initial kernel scaffold; baseline (speedup 1.0000x reference)
import jax
import jax.numpy as jnp
from jax import lax
from jax.experimental import pallas as pl
from jax.experimental.pallas import tpu as pltpu
from jax.experimental.pallas import tpu_sc as plsc

N_NODES = 10000
N_GENES = 128
N_EDGES = 320000
GC = 2 * N_GENES

NC = 2
NS = 16
NW = NC * NS
LANES = 16
PER_W = N_EDGES // NW
CHUNK = 200
NCHUNK = PER_W // CHUNK
NGRP = GC // LANES

ROW_BLK = 2000
NBLK = N_NODES // ROW_BLK


def _moments_body(yh_ref, yt_ref, ev_ref, mu_ref, w_ref, acc_ref, wacc_ref):
    i = pl.program_id(0)

    @pl.when(i == 0)
    def _():
        acc_ref[...] = jnp.zeros_like(acc_ref)
        wacc_ref[...] = jnp.zeros_like(wacc_ref)

    acc_ref[:, :N_GENES] += jnp.sum(yh_ref[...], axis=0, keepdims=True)
    acc_ref[:, N_GENES:] += jnp.sum(yt_ref[...], axis=0, keepdims=True)
    wacc_ref[0, 0] += jnp.sum(ev_ref[...])

    @pl.when(i == NBLK - 1)
    def _():
        mu_ref[...] = acc_ref[...] / N_NODES
        w_ref[...] = wacc_ref[...]


def _moments(y_hat, y_true, ev2d):
    return pl.pallas_call(
        _moments_body,
        grid=(NBLK,),
        in_specs=[
            pl.BlockSpec((ROW_BLK, N_GENES), lambda i: (i, 0)),
            pl.BlockSpec((ROW_BLK, N_GENES), lambda i: (i, 0)),
            pl.BlockSpec((ROW_BLK, N_EDGES // N_NODES), lambda i: (i, 0)),
        ],
        out_specs=[
            pl.BlockSpec((1, GC), lambda i: (0, 0)),
            pl.BlockSpec((1, 1), lambda i: (0, 0)),
        ],
        out_shape=[
            jax.ShapeDtypeStruct((1, GC), jnp.float32),
            jax.ShapeDtypeStruct((1, 1), jnp.float32),
        ],
        scratch_shapes=[
            pltpu.VMEM((1, GC), jnp.float32),
            pltpu.VMEM((1, 1), jnp.float32),
        ],
    )(y_hat, y_true, ev2d)


def _center_body(yh_ref, yt_ref, mu_ref, c_ref, den_ref, dacc_ref):
    i = pl.program_id(0)

    @pl.when(i == 0)
    def _():
        dacc_ref[...] = jnp.zeros_like(dacc_ref)

    ch = yh_ref[...] - mu_ref[0:1, :N_GENES]
    ct = yt_ref[...] - mu_ref[0:1, N_GENES:]
    c_ref[:, :N_GENES] = ch
    c_ref[:, N_GENES:] = ct
    dacc_ref[:, :N_GENES] += jnp.sum(ch * ch, axis=0, keepdims=True)
    dacc_ref[:, N_GENES:] += jnp.sum(ct * ct, axis=0, keepdims=True)

    @pl.when(i == NBLK - 1)
    def _():
        den_ref[...] = dacc_ref[...]


def _center(y_hat, y_true, mu):
    return pl.pallas_call(
        _center_body,
        grid=(NBLK,),
        in_specs=[
            pl.BlockSpec((ROW_BLK, N_GENES), lambda i: (i, 0)),
            pl.BlockSpec((ROW_BLK, N_GENES), lambda i: (i, 0)),
            pl.BlockSpec((1, GC), lambda i: (0, 0)),
        ],
        out_specs=[
            pl.BlockSpec((ROW_BLK, GC), lambda i: (i, 0)),
            pl.BlockSpec((1, GC), lambda i: (0, 0)),
        ],
        out_shape=[
            jax.ShapeDtypeStruct((N_NODES, GC), jnp.float32),
            jax.ShapeDtypeStruct((1, GC), jnp.float32),
        ],
        scratch_shapes=[pltpu.VMEM((1, GC), jnp.float32)],
    )(y_hat, y_true, mu)


def _edge_body(c_hbm, src_hbm, dst_hbm, w_hbm, out_hbm,
               idxs, idxd, wbuf, rows_s, rows_d, accbuf, sem_s, sem_d):
    wid = lax.axis_index("s") * NC + lax.axis_index("c")
    base0 = pl.multiple_of(wid * PER_W, 8)

    def chunk_body(c, accs):
        base = pl.multiple_of(base0 + c * CHUNK, 8)
        pltpu.sync_copy(src_hbm.at[pl.ds(base, CHUNK)], idxs)
        pltpu.sync_copy(dst_hbm.at[pl.ds(base, CHUNK)], idxd)
        pltpu.sync_copy(w_hbm.at[pl.ds(base, CHUNK)], wbuf)
        cp_s = pltpu.async_copy(c_hbm.at[idxs], rows_s, sem_s)
        cp_d = pltpu.async_copy(c_hbm.at[idxd], rows_d, sem_d)
        cp_s.wait()
        cp_d.wait()

        def edge_body(e, accs):
            wv = plsc.load_gather(wbuf, [jnp.zeros((LANES,), jnp.int32) + e])
            new = []
            for j in range(NGRP):
                s = rows_s[e, pl.ds(j * LANES, LANES)]
                d = rows_d[e, pl.ds(j * LANES, LANES)]
                new.append(accs[j] + s * d * wv)
            return tuple(new)

        return lax.fori_loop(0, CHUNK, edge_body, accs)

    accs = tuple(jnp.zeros((LANES,), jnp.float32) for _ in range(NGRP))
    accs = lax.fori_loop(0, NCHUNK, chunk_body, accs)
    for j in range(NGRP):
        accbuf[pl.ds(j * LANES, LANES)] = accs[j]
    pltpu.sync_copy(accbuf, out_hbm.at[wid])


def _edge_partials(c_cat, src, dst, edge_vals):
    mesh = plsc.VectorSubcoreMesh(
        core_axis_name="c", subcore_axis_name="s",
        num_cores=NC, num_subcores=NS)
    return pl.kernel(
        _edge_body,
        out_type=jax.ShapeDtypeStruct((NW, GC), jnp.float32),
        mesh=mesh,
        scratch_types=[
            pltpu.VMEM((CHUNK,), jnp.int32),
            pltpu.VMEM((CHUNK,), jnp.int32),
            pltpu.VMEM((CHUNK,), jnp.float32),
            pltpu.VMEM((CHUNK, GC), jnp.float32),
            pltpu.VMEM((CHUNK, GC), jnp.float32),
            pltpu.VMEM((GC,), jnp.float32),
            pltpu.SemaphoreType.DMA,
            pltpu.SemaphoreType.DMA,
        ],
    )(c_cat, src, dst, edge_vals)


def _final_body(p_ref, den_ref, w_ref, out_ref):
    num = jnp.sum(p_ref[...], axis=0, keepdims=True)
    den = den_ref[...]
    den = den + jnp.where(den == 0.0, 1e-6, 0.0)
    stats = (N_NODES / w_ref[0, 0]) * num / den
    diff = stats[0:1, :N_GENES] - stats[0:1, N_GENES:]
    out_ref[0, 0] = jnp.mean(diff * diff)


def _final(partials, den, w):
    return pl.pallas_call(
        _final_body,
        out_shape=jax.ShapeDtypeStruct((1, 1), jnp.float32),
    )(partials, den, w)


def kernel(Y_hat, Y_true, edge_index, edge_vals):
    ev2d = edge_vals.reshape(N_NODES, N_EDGES // N_NODES)
    mu, w = _moments(Y_hat, Y_true, ev2d)
    c_cat, den = _center(Y_hat, Y_true, mu)
    partials = _edge_partials(c_cat, edge_index[0], edge_index[1], edge_vals)
    loss = _final(partials, den, w)
    return loss[0, 0]


# speedup vs baseline: 5.8588x; 5.8588x over previous
import jax
import jax.numpy as jnp
from jax import lax
from jax.experimental import pallas as pl
from jax.experimental.pallas import tpu as pltpu
from jax.experimental.pallas import tpu_sc as plsc

N_NODES = 10000
N_GENES = 128
N_EDGES = 320000
GC = 2 * N_GENES

NC = 2
NS = 16
NW = NC * NS
LANES = 16
PER_W = N_EDGES // NW
CHUNK = 200
NCHUNK = PER_W // CHUNK
NGRP = GC // LANES

ROW_BLK = 2000
NBLK = N_NODES // ROW_BLK


def _moments_body(yh_ref, yt_ref, ev_ref, mu_ref, w_ref, acc_ref, wacc_ref):
    i = pl.program_id(0)

    @pl.when(i == 0)
    def _():
        acc_ref[...] = jnp.zeros_like(acc_ref)
        wacc_ref[...] = jnp.zeros_like(wacc_ref)

    acc_ref[:, :N_GENES] += jnp.sum(yh_ref[...], axis=0, keepdims=True)
    acc_ref[:, N_GENES:] += jnp.sum(yt_ref[...], axis=0, keepdims=True)
    wacc_ref[...] += jnp.sum(ev_ref[...])[None, None]

    @pl.when(i == NBLK - 1)
    def _():
        mu_ref[...] = acc_ref[...] / N_NODES
        w_ref[...] = wacc_ref[...]


def _moments(y_hat, y_true, ev2d):
    return pl.pallas_call(
        _moments_body,
        grid=(NBLK,),
        in_specs=[
            pl.BlockSpec((ROW_BLK, N_GENES), lambda i: (i, 0)),
            pl.BlockSpec((ROW_BLK, N_GENES), lambda i: (i, 0)),
            pl.BlockSpec((ROW_BLK, N_EDGES // N_NODES), lambda i: (i, 0)),
        ],
        out_specs=[
            pl.BlockSpec((1, GC), lambda i: (0, 0)),
            pl.BlockSpec((1, 1), lambda i: (0, 0)),
        ],
        out_shape=[
            jax.ShapeDtypeStruct((1, GC), jnp.float32),
            jax.ShapeDtypeStruct((1, 1), jnp.float32),
        ],
        scratch_shapes=[
            pltpu.VMEM((1, GC), jnp.float32),
            pltpu.VMEM((1, 1), jnp.float32),
        ],
    )(y_hat, y_true, ev2d)


def _center_body(yh_ref, yt_ref, mu_ref, c_ref, den_ref, dacc_ref):
    i = pl.program_id(0)

    @pl.when(i == 0)
    def _():
        dacc_ref[...] = jnp.zeros_like(dacc_ref)

    ch = yh_ref[...] - mu_ref[0:1, :N_GENES]
    ct = yt_ref[...] - mu_ref[0:1, N_GENES:]
    c_ref[:, :N_GENES] = ch
    c_ref[:, N_GENES:] = ct
    dacc_ref[:, :N_GENES] += jnp.sum(ch * ch, axis=0, keepdims=True)
    dacc_ref[:, N_GENES:] += jnp.sum(ct * ct, axis=0, keepdims=True)

    @pl.when(i == NBLK - 1)
    def _():
        den_ref[...] = dacc_ref[...]


def _center(y_hat, y_true, mu):
    return pl.pallas_call(
        _center_body,
        grid=(NBLK,),
        in_specs=[
            pl.BlockSpec((ROW_BLK, N_GENES), lambda i: (i, 0)),
            pl.BlockSpec((ROW_BLK, N_GENES), lambda i: (i, 0)),
            pl.BlockSpec((1, GC), lambda i: (0, 0)),
        ],
        out_specs=[
            pl.BlockSpec((ROW_BLK, GC), lambda i: (i, 0)),
            pl.BlockSpec((1, GC), lambda i: (0, 0)),
        ],
        out_shape=[
            jax.ShapeDtypeStruct((N_NODES, GC), jnp.float32),
            jax.ShapeDtypeStruct((1, GC), jnp.float32),
        ],
        scratch_shapes=[pltpu.VMEM((1, GC), jnp.float32)],
    )(y_hat, y_true, mu)


def _edge_body(c_hbm, src_hbm, dst_hbm, w_hbm, out_hbm,
               idxs, idxd, wbuf, rows_s, rows_d, accbuf, sem_s, sem_d):
    wid = lax.axis_index("s") * NC + lax.axis_index("c")
    base0 = pl.multiple_of(wid * PER_W, 8)

    def chunk_body(c, accs):
        base = pl.multiple_of(base0 + c * CHUNK, 8)
        pltpu.sync_copy(src_hbm.at[pl.ds(base, CHUNK)], idxs)
        pltpu.sync_copy(dst_hbm.at[pl.ds(base, CHUNK)], idxd)
        pltpu.sync_copy(w_hbm.at[pl.ds(base, CHUNK)], wbuf)
        cp_s = pltpu.async_copy(c_hbm.at[idxs], rows_s, sem_s)
        cp_d = pltpu.async_copy(c_hbm.at[idxd], rows_d, sem_d)
        cp_s.wait()
        cp_d.wait()

        def edge_body(e, accs):
            wv = plsc.load_gather(wbuf, [jnp.zeros((LANES,), jnp.int32) + e])
            new = []
            for j in range(NGRP):
                s = rows_s[e, pl.ds(j * LANES, LANES)]
                d = rows_d[e, pl.ds(j * LANES, LANES)]
                new.append(accs[j] + s * d * wv)
            return tuple(new)

        return lax.fori_loop(0, CHUNK, edge_body, accs)

    accs = tuple(jnp.zeros((LANES,), jnp.float32) for _ in range(NGRP))
    accs = lax.fori_loop(0, NCHUNK, chunk_body, accs)
    for j in range(NGRP):
        accbuf[pl.ds(j * LANES, LANES)] = accs[j]
    pltpu.sync_copy(accbuf, out_hbm.at[wid])


def _edge_partials(c_cat, src, dst, edge_vals):
    mesh = plsc.VectorSubcoreMesh(
        core_axis_name="c", subcore_axis_name="s",
        num_cores=NC, num_subcores=NS)
    return pl.kernel(
        _edge_body,
        out_type=jax.ShapeDtypeStruct((NW, GC), jnp.float32),
        mesh=mesh,
        compiler_params=pltpu.CompilerParams(needs_layout_passes=False),
        scratch_types=[
            pltpu.VMEM((CHUNK,), jnp.int32),
            pltpu.VMEM((CHUNK,), jnp.int32),
            pltpu.VMEM((CHUNK,), jnp.float32),
            pltpu.VMEM((CHUNK, GC), jnp.float32),
            pltpu.VMEM((CHUNK, GC), jnp.float32),
            pltpu.VMEM((GC,), jnp.float32),
            pltpu.SemaphoreType.DMA,
            pltpu.SemaphoreType.DMA,
        ],
    )(c_cat, src, dst, edge_vals)


def _final_body(p_ref, den_ref, w_ref, out_ref):
    num = jnp.sum(p_ref[...], axis=0, keepdims=True)
    den = den_ref[...]
    den = den + jnp.where(den == 0.0, 1e-6, 0.0)
    stats = (N_NODES / w_ref[0, 0]) * num / den
    diff = stats[0:1, :N_GENES] - stats[0:1, N_GENES:]
    out_ref[...] = jnp.mean(diff * diff)[None, None]


def _final(partials, den, w):
    return pl.pallas_call(
        _final_body,
        out_shape=jax.ShapeDtypeStruct((1, 1), jnp.float32),
    )(partials, den, w)


def kernel(Y_hat, Y_true, edge_index, edge_vals):
    ev2d = edge_vals.reshape(N_NODES, N_EDGES // N_NODES)
    mu, w = _moments(Y_hat, Y_true, ev2d)
    c_cat, den = _center(Y_hat, Y_true, mu)
    partials = _edge_partials(c_cat, edge_index[0], edge_index[1], edge_vals)
    loss = _final(partials, den, w)
    return loss[0, 0]


# speedup vs baseline: 9.3783x; 1.6007x over previous
import jax
import jax.numpy as jnp
from jax import lax
from jax.experimental import pallas as pl
from jax.experimental.pallas import tpu as pltpu
from jax.experimental.pallas import tpu_sc as plsc

N_NODES = 10000
N_GENES = 128
N_EDGES = 320000
GC = 2 * N_GENES

NC = 2
NS = 16
NW = NC * NS
LANES = 16
PER_W = N_EDGES // NW
CHUNK = 80
NCHUNK = PER_W // CHUNK
NGRP = GC // LANES

ROW_BLK = 2000
NBLK = N_NODES // ROW_BLK


def _moments_body(yh_ref, yt_ref, ev_ref, mu_ref, w_ref, acc_ref, wacc_ref):
    i = pl.program_id(0)

    @pl.when(i == 0)
    def _():
        acc_ref[...] = jnp.zeros_like(acc_ref)
        wacc_ref[...] = jnp.zeros_like(wacc_ref)

    acc_ref[:, :N_GENES] += jnp.sum(yh_ref[...], axis=0, keepdims=True)
    acc_ref[:, N_GENES:] += jnp.sum(yt_ref[...], axis=0, keepdims=True)
    wacc_ref[...] += jnp.sum(ev_ref[...])[None, None]

    @pl.when(i == NBLK - 1)
    def _():
        mu_ref[...] = acc_ref[...] / N_NODES
        w_ref[...] = wacc_ref[...]


def _moments(y_hat, y_true, ev2d):
    return pl.pallas_call(
        _moments_body,
        grid=(NBLK,),
        in_specs=[
            pl.BlockSpec((ROW_BLK, N_GENES), lambda i: (i, 0)),
            pl.BlockSpec((ROW_BLK, N_GENES), lambda i: (i, 0)),
            pl.BlockSpec((ROW_BLK, N_EDGES // N_NODES), lambda i: (i, 0)),
        ],
        out_specs=[
            pl.BlockSpec((1, GC), lambda i: (0, 0)),
            pl.BlockSpec((1, 1), lambda i: (0, 0)),
        ],
        out_shape=[
            jax.ShapeDtypeStruct((1, GC), jnp.float32),
            jax.ShapeDtypeStruct((1, 1), jnp.float32),
        ],
        scratch_shapes=[
            pltpu.VMEM((1, GC), jnp.float32),
            pltpu.VMEM((1, 1), jnp.float32),
        ],
    )(y_hat, y_true, ev2d)


def _center_body(yh_ref, yt_ref, mu_ref, c_ref, den_ref, dacc_ref):
    i = pl.program_id(0)

    @pl.when(i == 0)
    def _():
        dacc_ref[...] = jnp.zeros_like(dacc_ref)

    ch = yh_ref[...] - mu_ref[0:1, :N_GENES]
    ct = yt_ref[...] - mu_ref[0:1, N_GENES:]
    c_ref[:, :N_GENES] = ch
    c_ref[:, N_GENES:] = ct
    dacc_ref[:, :N_GENES] += jnp.sum(ch * ch, axis=0, keepdims=True)
    dacc_ref[:, N_GENES:] += jnp.sum(ct * ct, axis=0, keepdims=True)

    @pl.when(i == NBLK - 1)
    def _():
        den_ref[...] = dacc_ref[...]


def _center(y_hat, y_true, mu):
    return pl.pallas_call(
        _center_body,
        grid=(NBLK,),
        in_specs=[
            pl.BlockSpec((ROW_BLK, N_GENES), lambda i: (i, 0)),
            pl.BlockSpec((ROW_BLK, N_GENES), lambda i: (i, 0)),
            pl.BlockSpec((1, GC), lambda i: (0, 0)),
        ],
        out_specs=[
            pl.BlockSpec((ROW_BLK, GC), lambda i: (i, 0)),
            pl.BlockSpec((1, GC), lambda i: (0, 0)),
        ],
        out_shape=[
            jax.ShapeDtypeStruct((N_NODES, GC), jnp.float32),
            jax.ShapeDtypeStruct((1, GC), jnp.float32),
        ],
        scratch_shapes=[pltpu.VMEM((1, GC), jnp.float32)],
    )(y_hat, y_true, mu)


def _edge_body(c_hbm, src_hbm, dst_hbm, w_hbm, out_hbm,
               srcv, dstv, wb0, wb1, rs0, rs1, rd0, rd1, accbuf,
               sem0, sem1):
    wid = lax.axis_index("s") * NC + lax.axis_index("c")
    base0 = pl.multiple_of(wid * PER_W, 8)

    pltpu.sync_copy(src_hbm.at[pl.ds(base0, PER_W)], srcv)
    pltpu.sync_copy(dst_hbm.at[pl.ds(base0, PER_W)], dstv)

    bufs = ((rs0, rd0, wb0, sem0), (rs1, rd1, wb1, sem1))

    def issue(c, b):
        rs, rd, wb, sem = bufs[b]
        off = pl.multiple_of(c * CHUNK, 8)
        pltpu.async_copy(c_hbm.at[srcv.at[pl.ds(off, CHUNK)]], rs, sem)
        pltpu.async_copy(c_hbm.at[dstv.at[pl.ds(off, CHUNK)]], rd, sem)
        pltpu.async_copy(w_hbm.at[pl.ds(base0 + off, CHUNK)], wb, sem)

    def wait(b):
        rs, rd, wb, sem = bufs[b]
        pltpu.make_async_copy(c_hbm.at[srcv.at[pl.ds(0, CHUNK)]], rs, sem).wait()
        pltpu.make_async_copy(c_hbm.at[dstv.at[pl.ds(0, CHUNK)]], rd, sem).wait()
        pltpu.make_async_copy(w_hbm.at[pl.ds(base0, CHUNK)], wb, sem).wait()

    def compute(b, accs):
        rs, rd, wb, _ = bufs[b]

        def edge_body(e, accs):
            wv = plsc.load_gather(wb, [jnp.zeros((LANES,), jnp.int32) + e])
            new = []
            for j in range(NGRP):
                s = rs[e, pl.ds(j * LANES, LANES)]
                d = rd[e, pl.ds(j * LANES, LANES)]
                new.append(accs[j] + s * d * wv)
            return tuple(new)

        return lax.fori_loop(0, CHUNK, edge_body, accs)

    issue(0, 0)

    def pair_body(k, accs):
        c0 = 2 * k
        wait(0)
        issue(c0 + 1, 1)
        accs = compute(0, accs)
        wait(1)
        issue(c0 + 2, 0)
        return compute(1, accs)

    accs = tuple(jnp.zeros((LANES,), jnp.float32) for _ in range(NGRP))
    accs = lax.fori_loop(0, NCHUNK // 2, pair_body, accs)
    wait(0)
    accs = compute(0, accs)

    for j in range(NGRP):
        accbuf[pl.ds(j * LANES, LANES)] = accs[j]
    pltpu.sync_copy(accbuf, out_hbm.at[wid])


def _edge_partials(c_cat, src, dst, edge_vals):
    mesh = plsc.VectorSubcoreMesh(
        core_axis_name="c", subcore_axis_name="s",
        num_cores=NC, num_subcores=NS)
    return pl.kernel(
        _edge_body,
        out_type=jax.ShapeDtypeStruct((NW, GC), jnp.float32),
        mesh=mesh,
        compiler_params=pltpu.CompilerParams(needs_layout_passes=False),
        scratch_types=[
            pltpu.VMEM((PER_W,), jnp.int32),
            pltpu.VMEM((PER_W,), jnp.int32),
            pltpu.VMEM((CHUNK,), jnp.float32),
            pltpu.VMEM((CHUNK,), jnp.float32),
            pltpu.VMEM((CHUNK, GC), jnp.float32),
            pltpu.VMEM((CHUNK, GC), jnp.float32),
            pltpu.VMEM((CHUNK, GC), jnp.float32),
            pltpu.VMEM((CHUNK, GC), jnp.float32),
            pltpu.VMEM((GC,), jnp.float32),
            pltpu.SemaphoreType.DMA,
            pltpu.SemaphoreType.DMA,
        ],
    )(c_cat, src, dst, edge_vals)


def _final_body(p_ref, den_ref, w_ref, out_ref):
    num = jnp.sum(p_ref[...], axis=0, keepdims=True)
    den = den_ref[...]
    den = den + jnp.where(den == 0.0, 1e-6, 0.0)
    stats = (N_NODES / w_ref[0, 0]) * num / den
    diff = stats[0:1, :N_GENES] - stats[0:1, N_GENES:]
    out_ref[...] = jnp.mean(diff * diff)[None, None]


def _final(partials, den, w):
    return pl.pallas_call(
        _final_body,
        out_shape=jax.ShapeDtypeStruct((1, 1), jnp.float32),
    )(partials, den, w)


def kernel(Y_hat, Y_true, edge_index, edge_vals):
    ev2d = edge_vals.reshape(N_NODES, N_EDGES // N_NODES)
    mu, w = _moments(Y_hat, Y_true, ev2d)
    c_cat, den = _center(Y_hat, Y_true, mu)
    partials = _edge_partials(c_cat, edge_index[0], edge_index[1], edge_vals)
    loss = _final(partials, den, w)
    return loss[0, 0]


# speedup vs baseline: 9.6622x; 1.0303x over previous
import jax
import jax.numpy as jnp
import numpy as np
from jax import lax
from jax.experimental import pallas as pl
from jax.experimental.pallas import tpu as pltpu
from jax.experimental.pallas import tpu_sc as plsc

N_NODES = 10000
N_GENES = 128
N_EDGES = 320000
GC = 2 * N_GENES

NC = 2
NS = 16
NW = NC * NS
LANES = 16
PER_W = N_EDGES // NW
CHUNK = 80
NCHUNK = PER_W // CHUNK
NGRP = GC // LANES
GW = GC // 2

ROW_BLK = 2000
NBLK = N_NODES // ROW_BLK


def _moments_body(yh_ref, yt_ref, ev_ref, mu_ref, w_ref, acc_ref, wacc_ref):
    i = pl.program_id(0)

    @pl.when(i == 0)
    def _():
        acc_ref[...] = jnp.zeros_like(acc_ref)
        wacc_ref[...] = jnp.zeros_like(wacc_ref)

    acc_ref[:, :N_GENES] += jnp.sum(yh_ref[...], axis=0, keepdims=True)
    acc_ref[:, N_GENES:] += jnp.sum(yt_ref[...], axis=0, keepdims=True)
    wacc_ref[...] += jnp.sum(ev_ref[...])[None, None]

    @pl.when(i == NBLK - 1)
    def _():
        mu_ref[...] = acc_ref[...] / N_NODES
        w_ref[...] = wacc_ref[...]


def _moments(y_hat, y_true, ev2d):
    return pl.pallas_call(
        _moments_body,
        grid=(NBLK,),
        in_specs=[
            pl.BlockSpec((ROW_BLK, N_GENES), lambda i: (i, 0)),
            pl.BlockSpec((ROW_BLK, N_GENES), lambda i: (i, 0)),
            pl.BlockSpec((ROW_BLK, N_EDGES // N_NODES), lambda i: (i, 0)),
        ],
        out_specs=[
            pl.BlockSpec((1, GC), lambda i: (0, 0)),
            pl.BlockSpec((1, 1), lambda i: (0, 0)),
        ],
        out_shape=[
            jax.ShapeDtypeStruct((1, GC), jnp.float32),
            jax.ShapeDtypeStruct((1, 1), jnp.float32),
        ],
        scratch_shapes=[
            pltpu.VMEM((1, GC), jnp.float32),
            pltpu.VMEM((1, 1), jnp.float32),
        ],
    )(y_hat, y_true, ev2d)


def _center_body(yh_ref, yt_ref, mu_ref, c_ref, den_ref, dacc_ref):
    i = pl.program_id(0)

    @pl.when(i == 0)
    def _():
        dacc_ref[...] = jnp.zeros_like(dacc_ref)

    ch = yh_ref[...] - mu_ref[0:1, :N_GENES]
    ct = yt_ref[...] - mu_ref[0:1, N_GENES:]
    c_ref[:, :N_GENES] = ch.astype(jnp.bfloat16)
    c_ref[:, N_GENES:] = ct.astype(jnp.bfloat16)
    dacc_ref[:, :N_GENES] += jnp.sum(ch * ch, axis=0, keepdims=True)
    dacc_ref[:, N_GENES:] += jnp.sum(ct * ct, axis=0, keepdims=True)

    @pl.when(i == NBLK - 1)
    def _():
        den_ref[...] = dacc_ref[...]


def _center(y_hat, y_true, mu):
    return pl.pallas_call(
        _center_body,
        grid=(NBLK,),
        in_specs=[
            pl.BlockSpec((ROW_BLK, N_GENES), lambda i: (i, 0)),
            pl.BlockSpec((ROW_BLK, N_GENES), lambda i: (i, 0)),
            pl.BlockSpec((1, GC), lambda i: (0, 0)),
        ],
        out_specs=[
            pl.BlockSpec((ROW_BLK, GC), lambda i: (i, 0)),
            pl.BlockSpec((1, GC), lambda i: (0, 0)),
        ],
        out_shape=[
            jax.ShapeDtypeStruct((N_NODES, GC), jnp.bfloat16),
            jax.ShapeDtypeStruct((1, GC), jnp.float32),
        ],
        scratch_shapes=[pltpu.VMEM((1, GC), jnp.float32)],
    )(y_hat, y_true, mu)


def _edge_body(c_hbm, src_hbm, dst_hbm, w_hbm, out_hbm,
               srcv, dstv, wb0, wb1, rs0, rs1, rd0, rd1, accbuf,
               sem0, sem1):
    wid = lax.axis_index("s") * NC + lax.axis_index("c")
    base0 = pl.multiple_of(wid * PER_W, 8)

    pltpu.sync_copy(src_hbm.at[pl.ds(base0, PER_W)], srcv)
    pltpu.sync_copy(dst_hbm.at[pl.ds(base0, PER_W)], dstv)

    bufs = ((rs0, rd0, wb0, sem0), (rs1, rd1, wb1, sem1))

    def issue(c, b):
        rs, rd, wb, sem = bufs[b]
        off = pl.multiple_of(c * CHUNK, 8)
        pltpu.async_copy(c_hbm.at[srcv.at[pl.ds(off, CHUNK)]], rs, sem)
        pltpu.async_copy(c_hbm.at[dstv.at[pl.ds(off, CHUNK)]], rd, sem)
        pltpu.async_copy(w_hbm.at[pl.ds(base0 + off, CHUNK)], wb, sem)

    def wait(b):
        rs, rd, wb, sem = bufs[b]
        pltpu.make_async_copy(c_hbm.at[srcv.at[pl.ds(0, CHUNK)]], rs, sem).wait()
        pltpu.make_async_copy(c_hbm.at[dstv.at[pl.ds(0, CHUNK)]], rd, sem).wait()
        pltpu.make_async_copy(w_hbm.at[pl.ds(base0, CHUNK)], wb, sem).wait()

    def compute(b, accs):
        rs, rd, wb, _ = bufs[b]

        def edge_body(e, accs):
            wv = plsc.load_gather(wb, [jnp.zeros((LANES,), jnp.int32) + e])
            new = []
            for j in range(NGRP // 2):
                s2 = plsc.bitcast(rs[e, pl.ds(j * LANES, LANES)], jnp.bfloat16)
                d2 = plsc.bitcast(rd[e, pl.ds(j * LANES, LANES)], jnp.bfloat16)
                sa, sb = plsc.unpack(s2, format=plsc.PackFormat.INTERLEAVED)
                da, db = plsc.unpack(d2, format=plsc.PackFormat.INTERLEAVED)
                new.append(accs[2 * j] + sa * da * wv)
                new.append(accs[2 * j + 1] + sb * db * wv)
            return tuple(new)

        return lax.fori_loop(0, CHUNK, edge_body, accs)

    issue(0, 0)

    def pair_body(k, accs):
        c0 = 2 * k
        wait(0)
        issue(c0 + 1, 1)
        accs = compute(0, accs)
        wait(1)
        issue(c0 + 2, 0)
        return compute(1, accs)

    accs = tuple(jnp.zeros((LANES,), jnp.float32) for _ in range(NGRP))
    accs = lax.fori_loop(0, NCHUNK // 2, pair_body, accs)
    wait(0)
    accs = compute(0, accs)

    for j in range(NGRP):
        accbuf[pl.ds(j * LANES, LANES)] = accs[j]
    pltpu.sync_copy(accbuf, out_hbm.at[wid])


def _edge_partials(c_cat, src, dst, edge_vals):
    mesh = plsc.VectorSubcoreMesh(
        core_axis_name="c", subcore_axis_name="s",
        num_cores=NC, num_subcores=NS)
    return pl.kernel(
        _edge_body,
        out_type=jax.ShapeDtypeStruct((NW, GC), jnp.float32),
        mesh=mesh,
        compiler_params=pltpu.CompilerParams(needs_layout_passes=False),
        scratch_types=[
            pltpu.VMEM((PER_W,), jnp.int32),
            pltpu.VMEM((PER_W,), jnp.int32),
            pltpu.VMEM((CHUNK,), jnp.float32),
            pltpu.VMEM((CHUNK,), jnp.float32),
            pltpu.VMEM((CHUNK, GW), jnp.int32),
            pltpu.VMEM((CHUNK, GW), jnp.int32),
            pltpu.VMEM((CHUNK, GW), jnp.int32),
            pltpu.VMEM((CHUNK, GW), jnp.int32),
            pltpu.VMEM((GC,), jnp.float32),
            pltpu.SemaphoreType.DMA,
            pltpu.SemaphoreType.DMA,
        ],
    )(c_cat, src, dst, edge_vals)


def _final_body(p_ref, den_ref, w_ref, out_ref):
    num = jnp.sum(p_ref[...], axis=0, keepdims=True)
    den = den_ref[...]
    den = den + jnp.where(den == 0.0, 1e-6, 0.0)
    stats = (N_NODES / w_ref[0, 0]) * num / den
    diff = stats[0:1, :N_GENES] - stats[0:1, N_GENES:]
    out_ref[...] = jnp.mean(diff * diff)[None, None]


def _final(partials, den, w):
    return pl.pallas_call(
        _final_body,
        out_shape=jax.ShapeDtypeStruct((1, 1), jnp.float32),
    )(partials, den, w)


_POS = np.array(
    [32 * (g // 32) + (g % 32) // 2 + 16 * (g % 2) for g in range(GC)],
    dtype=np.int32)


def kernel(Y_hat, Y_true, edge_index, edge_vals):
    ev2d = edge_vals.reshape(N_NODES, N_EDGES // N_NODES)
    mu, w = _moments(Y_hat, Y_true, ev2d)
    c_cat, den = _center(Y_hat, Y_true, mu)
    c32 = lax.bitcast_convert_type(
        c_cat.reshape(N_NODES, GW, 2), jnp.int32)
    partials = _edge_partials(c32, edge_index[0], edge_index[1], edge_vals)
    loss = _final(partials[:, _POS], den, w)
    return loss[0, 0]


# speedup vs baseline: 9.6705x; 1.0009x over previous
import jax
import jax.numpy as jnp
import numpy as np
from jax import lax
from jax.experimental import pallas as pl
from jax.experimental.pallas import tpu as pltpu
from jax.experimental.pallas import tpu_sc as plsc

N_NODES = 10000
N_GENES = 128
N_EDGES = 320000
GC = 2 * N_GENES

NC = 2
NS = 16
NW = NC * NS
LANES = 16
PER_W = N_EDGES // NW
CHUNK = 80
NCHUNK = PER_W // CHUNK
NGRP = GC // LANES
GW = GC // 2

ROW_BLK = 2000
NBLK = N_NODES // ROW_BLK


def _moments_body(yh_ref, yt_ref, ev_ref, mu_ref, w_ref, acc_ref, wacc_ref):
    i = pl.program_id(0)

    @pl.when(i == 0)
    def _():
        acc_ref[...] = jnp.zeros_like(acc_ref)
        wacc_ref[...] = jnp.zeros_like(wacc_ref)

    acc_ref[:, :N_GENES] += jnp.sum(yh_ref[...], axis=0, keepdims=True)
    acc_ref[:, N_GENES:] += jnp.sum(yt_ref[...], axis=0, keepdims=True)
    wacc_ref[...] += jnp.sum(ev_ref[...])[None, None]

    @pl.when(i == NBLK - 1)
    def _():
        mu_ref[...] = acc_ref[...] / N_NODES
        w_ref[...] = wacc_ref[...]


def _moments(y_hat, y_true, ev2d):
    return pl.pallas_call(
        _moments_body,
        grid=(NBLK,),
        in_specs=[
            pl.BlockSpec((ROW_BLK, N_GENES), lambda i: (i, 0)),
            pl.BlockSpec((ROW_BLK, N_GENES), lambda i: (i, 0)),
            pl.BlockSpec((ROW_BLK, N_EDGES // N_NODES), lambda i: (i, 0)),
        ],
        out_specs=[
            pl.BlockSpec((1, GC), lambda i: (0, 0)),
            pl.BlockSpec((1, 1), lambda i: (0, 0)),
        ],
        out_shape=[
            jax.ShapeDtypeStruct((1, GC), jnp.float32),
            jax.ShapeDtypeStruct((1, 1), jnp.float32),
        ],
        scratch_shapes=[
            pltpu.VMEM((1, GC), jnp.float32),
            pltpu.VMEM((1, 1), jnp.float32),
        ],
    )(y_hat, y_true, ev2d)


def _center_body(yh_ref, yt_ref, mu_ref, c_ref, den_ref, dacc_ref):
    i = pl.program_id(0)

    @pl.when(i == 0)
    def _():
        dacc_ref[...] = jnp.zeros_like(dacc_ref)

    ch = yh_ref[...] - mu_ref[0:1, :N_GENES]
    ct = yt_ref[...] - mu_ref[0:1, N_GENES:]
    c_ref[:, :N_GENES] = ch.astype(jnp.bfloat16)
    c_ref[:, N_GENES:] = ct.astype(jnp.bfloat16)
    dacc_ref[:, :N_GENES] += jnp.sum(ch * ch, axis=0, keepdims=True)
    dacc_ref[:, N_GENES:] += jnp.sum(ct * ct, axis=0, keepdims=True)

    @pl.when(i == NBLK - 1)
    def _():
        den_ref[...] = dacc_ref[...]


def _center(y_hat, y_true, mu):
    return pl.pallas_call(
        _center_body,
        grid=(NBLK,),
        in_specs=[
            pl.BlockSpec((ROW_BLK, N_GENES), lambda i: (i, 0)),
            pl.BlockSpec((ROW_BLK, N_GENES), lambda i: (i, 0)),
            pl.BlockSpec((1, GC), lambda i: (0, 0)),
        ],
        out_specs=[
            pl.BlockSpec((ROW_BLK, GC), lambda i: (i, 0)),
            pl.BlockSpec((1, GC), lambda i: (0, 0)),
        ],
        out_shape=[
            jax.ShapeDtypeStruct((N_NODES, GC), jnp.bfloat16),
            jax.ShapeDtypeStruct((1, GC), jnp.float32),
        ],
        scratch_shapes=[pltpu.VMEM((1, GC), jnp.float32)],
    )(y_hat, y_true, mu)


def _edge_body(c_hbm, src_hbm, dst_hbm, w_hbm, out_hbm,
               srcv, dstv, wb0, wb1, rs0, rs1, rd0, rd1, accbuf,
               sem0, sem1):
    wid = lax.axis_index("s") * NC + lax.axis_index("c")
    base0 = pl.multiple_of(wid * PER_W, 8)

    pltpu.sync_copy(src_hbm.at[pl.ds(base0, PER_W)], srcv)
    pltpu.sync_copy(dst_hbm.at[pl.ds(base0, PER_W)], dstv)

    bufs = ((rs0, rd0, wb0, sem0), (rs1, rd1, wb1, sem1))

    def issue(c, b):
        rs, rd, wb, sem = bufs[b]
        off = pl.multiple_of(c * CHUNK, 8)
        pltpu.async_copy(c_hbm.at[srcv.at[pl.ds(off, CHUNK)]], rs, sem)
        pltpu.async_copy(c_hbm.at[dstv.at[pl.ds(off, CHUNK)]], rd, sem)
        pltpu.async_copy(w_hbm.at[pl.ds(base0 + off, CHUNK)], wb, sem)

    def wait(b):
        rs, rd, wb, sem = bufs[b]
        pltpu.make_async_copy(c_hbm.at[srcv.at[pl.ds(0, CHUNK)]], rs, sem).wait()
        pltpu.make_async_copy(c_hbm.at[dstv.at[pl.ds(0, CHUNK)]], rd, sem).wait()
        pltpu.make_async_copy(w_hbm.at[pl.ds(base0, CHUNK)], wb, sem).wait()

    def compute(b, accs):
        rs, rd, wb, _ = bufs[b]

        def one_edge(e, accs):
            wv = plsc.load_gather(wb, [jnp.zeros((LANES,), jnp.int32) + e])
            wpk = plsc.pack(wv, wv, format=plsc.PackFormat.INTERLEAVED)
            new = []
            for j in range(NGRP // 2):
                s2 = plsc.bitcast(rs[e, pl.ds(j * LANES, LANES)], jnp.bfloat16)
                d2 = plsc.bitcast(rd[e, pl.ds(j * LANES, LANES)], jnp.bfloat16)
                pa, pb = plsc.unpack(
                    s2 * d2 * wpk, format=plsc.PackFormat.INTERLEAVED)
                new.append(accs[2 * j] + pa)
                new.append(accs[2 * j + 1] + pb)
            return tuple(new)

        def edge_body(h, accs):
            accs = one_edge(2 * h, accs)
            return one_edge(2 * h + 1, accs)

        return lax.fori_loop(0, CHUNK // 2, edge_body, accs)

    issue(0, 0)

    def pair_body(k, accs):
        c0 = 2 * k
        wait(0)
        issue(c0 + 1, 1)
        accs = compute(0, accs)
        wait(1)
        issue(c0 + 2, 0)
        return compute(1, accs)

    accs = tuple(jnp.zeros((LANES,), jnp.float32) for _ in range(NGRP))
    accs = lax.fori_loop(0, NCHUNK // 2, pair_body, accs)
    wait(0)
    accs = compute(0, accs)

    for j in range(NGRP):
        accbuf[pl.ds(j * LANES, LANES)] = accs[j]
    pltpu.sync_copy(accbuf, out_hbm.at[wid])


def _edge_partials(c_cat, src, dst, edge_vals):
    mesh = plsc.VectorSubcoreMesh(
        core_axis_name="c", subcore_axis_name="s",
        num_cores=NC, num_subcores=NS)
    return pl.kernel(
        _edge_body,
        out_type=jax.ShapeDtypeStruct((NW, GC), jnp.float32),
        mesh=mesh,
        compiler_params=pltpu.CompilerParams(needs_layout_passes=False),
        scratch_types=[
            pltpu.VMEM((PER_W,), jnp.int32),
            pltpu.VMEM((PER_W,), jnp.int32),
            pltpu.VMEM((CHUNK,), jnp.float32),
            pltpu.VMEM((CHUNK,), jnp.float32),
            pltpu.VMEM((CHUNK, GW), jnp.int32),
            pltpu.VMEM((CHUNK, GW), jnp.int32),
            pltpu.VMEM((CHUNK, GW), jnp.int32),
            pltpu.VMEM((CHUNK, GW), jnp.int32),
            pltpu.VMEM((GC,), jnp.float32),
            pltpu.SemaphoreType.DMA,
            pltpu.SemaphoreType.DMA,
        ],
    )(c_cat, src, dst, edge_vals)


def _final_body(p_ref, den_ref, w_ref, out_ref):
    num = jnp.sum(p_ref[...], axis=0, keepdims=True)
    den = den_ref[...]
    den = den + jnp.where(den == 0.0, 1e-6, 0.0)
    stats = (N_NODES / w_ref[0, 0]) * num / den
    diff = stats[0:1, :N_GENES] - stats[0:1, N_GENES:]
    out_ref[...] = jnp.mean(diff * diff)[None, None]


def _final(partials, den, w):
    return pl.pallas_call(
        _final_body,
        out_shape=jax.ShapeDtypeStruct((1, 1), jnp.float32),
    )(partials, den, w)


_POS = np.array(
    [32 * (g // 32) + (g % 32) // 2 + 16 * (g % 2) for g in range(GC)],
    dtype=np.int32)


def kernel(Y_hat, Y_true, edge_index, edge_vals):
    ev2d = edge_vals.reshape(N_NODES, N_EDGES // N_NODES)
    mu, w = _moments(Y_hat, Y_true, ev2d)
    c_cat, den = _center(Y_hat, Y_true, mu)
    c32 = lax.bitcast_convert_type(
        c_cat.reshape(N_NODES, GW, 2), jnp.int32)
    partials = _edge_partials(c32, edge_index[0], edge_index[1], edge_vals)
    loss = _final(partials[:, _POS], den, w)
    return loss[0, 0]
